# Initial kernel scaffold; baseline (speedup 1.0000x reference)
import jax
import jax.numpy as jnp
from jax.experimental import pallas as pl


def kernel(table, input):
    raise NotImplementedError("write your pallas kernel here")


# speedup vs baseline: 3.0610x; 3.0610x over previous
import functools

import jax
import jax.numpy as jnp
from jax import lax
from jax.experimental import pallas as pl
from jax.experimental.pallas import tpu as pltpu
from jax.experimental.pallas import tpu_sc as plsc

_NC, _NS = 2, 16
_NW = _NC * _NS
_CHUNK = 128


@functools.partial(jax.jit, static_argnums=(2, 3, 4))
def _embedding_lookup(tpad, idx2d, n, dp, nch):
    per_w = n // _NW
    mesh = plsc.VectorSubcoreMesh(core_axis_name="c", subcore_axis_name="s")

    @functools.partial(
        pl.kernel,
        out_type=jax.ShapeDtypeStruct((n, dp), tpad.dtype),
        mesh=mesh,
        compiler_params=pltpu.CompilerParams(use_tc_tiling_on_sc=False),
        scratch_types=[
            pltpu.VMEM((nch, _CHUNK), jnp.int32),
            pltpu.VMEM((_CHUNK, dp), jnp.float32),
            pltpu.SemaphoreType.DMA,
        ],
    )
    def body(table_hbm, idx_hbm, out_hbm, idx_v, buf, gsem):
        wid = lax.axis_index("s") * _NC + lax.axis_index("c")
        base = wid * per_w

        pltpu.sync_copy(idx_hbm.at[pl.ds(wid * nch, nch)], idx_v)

        @pl.loop(0, nch)
        def _(j):
            pltpu.async_copy(table_hbm.at[idx_v.at[j]], buf, gsem).wait()
            pltpu.sync_copy(buf, out_hbm.at[pl.ds(base + j * _CHUNK, _CHUNK)])

    return body(tpad, idx2d)


def kernel(table, input):
    bsz, seq = input.shape
    v, d = table.shape
    dp = (d + 7) // 8 * 8
    n = bsz * seq
    tpad = jnp.pad(table, ((0, 0), (0, dp - d)))
    idx2d = input.reshape(n // _CHUNK, _CHUNK).astype(jnp.int32)
    nch = (n // _NW) // _CHUNK
    out = _embedding_lookup(tpad, idx2d, n, dp, nch)
    return out[:, :d].reshape(bsz, seq, d)


# speedup vs baseline: 6.1568x; 2.0114x over previous
import functools

import jax
import jax.numpy as jnp
from jax import lax
from jax.experimental import pallas as pl
from jax.experimental.pallas import tpu as pltpu
from jax.experimental.pallas import tpu_sc as plsc

_NC, _NS = 2, 16
_NW = _NC * _NS
_CHUNK = 128
_WIDE = 128
_NBUF = 2


@functools.partial(jax.jit, static_argnums=(2, 3, 4))
def _embedding_lookup(tpad, idx2d, n, dp, nch):
    per_w = n // _NW
    mesh = plsc.VectorSubcoreMesh(core_axis_name="c", subcore_axis_name="s")

    @functools.partial(
        pl.kernel,
        out_type=jax.ShapeDtypeStruct((n, _WIDE), tpad.dtype),
        mesh=mesh,
        compiler_params=pltpu.CompilerParams(use_tc_tiling_on_sc=False),
        scratch_types=[
            pltpu.VMEM((nch, _CHUNK), jnp.int32),
            pltpu.VMEM((_CHUNK, dp), jnp.float32),
            pltpu.VMEM((_CHUNK, dp), jnp.float32),
            pltpu.SemaphoreType.DMA,
            pltpu.SemaphoreType.DMA,
            pltpu.SemaphoreType.DMA,
            pltpu.SemaphoreType.DMA,
        ],
    )
    def body(table_hbm, idx_hbm, out_hbm, idx_v, buf0, buf1, g0, g1, o0, o1):
        wid = lax.axis_index("s") * _NC + lax.axis_index("c")
        base = wid * per_w
        bufs = (buf0, buf1)
        gsems = (g0, g1)
        osems = (o0, o1)

        def out_slab(j):
            return out_hbm.at[pl.ds(base + j * _CHUNK, _CHUNK), pl.ds(0, dp)]

        pltpu.sync_copy(idx_hbm.at[pl.ds(wid * nch, nch)], idx_v)

        for b in range(_NBUF):
            pltpu.async_copy(table_hbm.at[idx_v.at[b]], bufs[b], gsems[b])

        steps = nch // _NBUF

        @pl.loop(0, steps)
        def _(g):
            j0 = g * _NBUF
            for b in range(_NBUF):
                pltpu.make_async_copy(
                    table_hbm.at[idx_v.at[j0 + b]], bufs[b], gsems[b]).wait()
                pltpu.async_copy(bufs[b], out_slab(j0 + b), osems[b])
            for b in range(_NBUF):

                @pl.when(g < steps - 1)
                def _():
                    pltpu.make_async_copy(
                        bufs[b], out_slab(j0 + b), osems[b]).wait()
                    pltpu.async_copy(
                        table_hbm.at[idx_v.at[j0 + b + _NBUF]],
                        bufs[b], gsems[b])

        for b in range(_NBUF):
            pltpu.make_async_copy(
                bufs[b], out_slab(nch - _NBUF + b), osems[b]).wait()

    return body(tpad, idx2d)


def kernel(table, input):
    bsz, seq = input.shape
    v, d = table.shape
    dp = (d + 7) // 8 * 8
    n = bsz * seq
    tpad = jnp.pad(table, ((0, 0), (0, dp - d)))
    idx2d = input.reshape(n // _CHUNK, _CHUNK).astype(jnp.int32)
    nch = (n // _NW) // _CHUNK
    out = _embedding_lookup(tpad, idx2d, n, dp, nch)
    return out[:, :d].reshape(bsz, seq, d)
